# baseline (device time: 212381 ns/iter reference)
import jax
import jax.numpy as jnp
from jax import lax
from jax.experimental import pallas as pl
from jax.experimental.pallas import tpu as pltpu

N_DEV = 16


def kernel(x, w_mat, scale_x, scale_w):
    k, n = w_mat.shape
    m = x.shape[0]
    m_per = m // N_DEV
    n_hops = N_DEV - 1

    def body(x_ref, w_ref, sx_ref, sw_ref, out_ref,
             send_ref, comm_ref, send_sems, recv_sems):
        my = lax.axis_index("i")
        left = lax.rem(my + N_DEV - 1, N_DEV)
        right = lax.rem(my + 1, N_DEV)

        barrier_sem = pltpu.get_barrier_semaphore()
        for nbr in (left, right):
            pl.semaphore_signal(barrier_sem, inc=1, device_id=(nbr,),
                                device_id_type=pl.DeviceIdType.MESH)
        pl.semaphore_wait(barrier_sem, 2)

        wb = w_ref[:, :].astype(jnp.bfloat16)

        def partial_chunk(c):
            xb = x_ref[pl.ds(c * m_per, m_per), :].astype(jnp.bfloat16)
            return jnp.dot(xb, wb, preferred_element_type=jnp.float32)

        send_ref[0, :, :] = partial_chunk(
            lax.rem(my + N_DEV - 1, N_DEV)).astype(jnp.bfloat16)

        for s in range(n_hops):
            rdma = pltpu.make_async_remote_copy(
                src_ref=send_ref.at[s % 2],
                dst_ref=comm_ref.at[s],
                send_sem=send_sems.at[s],
                recv_sem=recv_sems.at[s],
                device_id=(right,),
                device_id_type=pl.DeviceIdType.MESH,
            )
            rdma.start()
            rdma.wait()

            c_recv = lax.rem(my + 2 * N_DEV - 2 - s, N_DEV)
            acc = comm_ref[s, :, :].astype(jnp.float32) + partial_chunk(c_recv)
            if s < n_hops - 1:
                send_ref[(s + 1) % 2, :, :] = acc.astype(jnp.bfloat16)
            else:
                scale = sx_ref[0] * sw_ref[0]
                out_ref[:, :] = jnp.maximum(acc * scale, 0.0)

    return pl.pallas_call(
        body,
        out_shape=jax.ShapeDtypeStruct((m_per, n), jnp.float32),
        in_specs=[
            pl.BlockSpec(memory_space=pltpu.VMEM),
            pl.BlockSpec(memory_space=pltpu.VMEM),
            pl.BlockSpec(memory_space=pltpu.SMEM),
            pl.BlockSpec(memory_space=pltpu.SMEM),
        ],
        out_specs=pl.BlockSpec(memory_space=pltpu.VMEM),
        scratch_shapes=[
            pltpu.VMEM((2, m_per, n), jnp.bfloat16),
            pltpu.VMEM((n_hops, m_per, n), jnp.bfloat16),
            pltpu.SemaphoreType.DMA((n_hops,)),
            pltpu.SemaphoreType.DMA((n_hops,)),
        ],
        compiler_params=pltpu.CompilerParams(collective_id=0),
    )(x, w_mat, scale_x, scale_w)


# device time: 148415 ns/iter; 1.4310x vs baseline; 1.4310x over previous
import jax
import jax.numpy as jnp
from jax import lax
from jax.experimental import pallas as pl
from jax.experimental.pallas import tpu as pltpu

N_DEV = 16


def kernel(x, w_mat, scale_x, scale_w):
    k, n = w_mat.shape
    m = x.shape[0]
    m_per = m // N_DEV
    nh = n // 2
    n_hops = N_DEV - 1

    def body(x_ref, w_ref, sx_ref, sw_ref, out_ref,
             send_a, send_b, comm_a, comm_b,
             send_sems_a, recv_sems_a, send_sems_b, recv_sems_b):
        my = lax.axis_index("i")
        left = lax.rem(my + N_DEV - 1, N_DEV)
        right = lax.rem(my + 1, N_DEV)

        barrier_sem = pltpu.get_barrier_semaphore()
        for nbr in (left, right):
            pl.semaphore_signal(barrier_sem, inc=1, device_id=(nbr,),
                                device_id_type=pl.DeviceIdType.MESH)
        pl.semaphore_wait(barrier_sem, 2)

        wa = w_ref[:, :nh].astype(jnp.bfloat16)
        wb = w_ref[:, nh:].astype(jnp.bfloat16)

        def xblk(c):
            return x_ref[pl.ds(c * m_per, m_per), :].astype(jnp.bfloat16)

        def partial_a(c):
            return jnp.dot(xblk(c), wa, preferred_element_type=jnp.float32)

        def partial_b(c):
            return jnp.dot(xblk(c), wb, preferred_element_type=jnp.float32)

        send_a[0, :, :] = partial_a(
            lax.rem(my + N_DEV - 1, N_DEV)).astype(jnp.bfloat16)
        send_b[0, :, :] = partial_b(
            lax.rem(my + 1, N_DEV)).astype(jnp.bfloat16)

        for s in range(n_hops):
            rdma_a = pltpu.make_async_remote_copy(
                src_ref=send_a.at[s % 2],
                dst_ref=comm_a.at[s],
                send_sem=send_sems_a.at[s],
                recv_sem=recv_sems_a.at[s],
                device_id=(right,),
                device_id_type=pl.DeviceIdType.MESH,
            )
            rdma_b = pltpu.make_async_remote_copy(
                src_ref=send_b.at[s % 2],
                dst_ref=comm_b.at[s],
                send_sem=send_sems_b.at[s],
                recv_sem=recv_sems_b.at[s],
                device_id=(left,),
                device_id_type=pl.DeviceIdType.MESH,
            )
            rdma_a.start()
            rdma_b.start()
            rdma_a.wait()
            rdma_b.wait()

            ca = lax.rem(my + 2 * N_DEV - 2 - s, N_DEV)
            cb = lax.rem(my + 2 + s, N_DEV)
            acc_a = comm_a[s, :, :].astype(jnp.float32) + partial_a(ca)
            acc_b = comm_b[s, :, :].astype(jnp.float32) + partial_b(cb)
            if s < n_hops - 1:
                send_a[(s + 1) % 2, :, :] = acc_a.astype(jnp.bfloat16)
                send_b[(s + 1) % 2, :, :] = acc_b.astype(jnp.bfloat16)
            else:
                scale = sx_ref[0] * sw_ref[0]
                out_ref[:, :nh] = jnp.maximum(acc_a * scale, 0.0)
                out_ref[:, nh:] = jnp.maximum(acc_b * scale, 0.0)

    return pl.pallas_call(
        body,
        out_shape=jax.ShapeDtypeStruct((m_per, n), jnp.float32),
        in_specs=[
            pl.BlockSpec(memory_space=pltpu.VMEM),
            pl.BlockSpec(memory_space=pltpu.VMEM),
            pl.BlockSpec(memory_space=pltpu.SMEM),
            pl.BlockSpec(memory_space=pltpu.SMEM),
        ],
        out_specs=pl.BlockSpec(memory_space=pltpu.VMEM),
        scratch_shapes=[
            pltpu.VMEM((2, m_per, nh), jnp.bfloat16),
            pltpu.VMEM((2, m_per, nh), jnp.bfloat16),
            pltpu.VMEM((n_hops, m_per, nh), jnp.bfloat16),
            pltpu.VMEM((n_hops, m_per, nh), jnp.bfloat16),
            pltpu.SemaphoreType.DMA((n_hops,)),
            pltpu.SemaphoreType.DMA((n_hops,)),
            pltpu.SemaphoreType.DMA((n_hops,)),
            pltpu.SemaphoreType.DMA((n_hops,)),
        ],
        compiler_params=pltpu.CompilerParams(collective_id=0),
    )(x, w_mat, scale_x, scale_w)


# device time: 98592 ns/iter; 2.1541x vs baseline; 1.5053x over previous
import jax
import jax.numpy as jnp
from jax import lax
from jax.experimental import pallas as pl
from jax.experimental.pallas import tpu as pltpu

N_DEV = 16
NSUB = 2
N_HOPS = N_DEV - 1


def kernel(x, w_mat, scale_x, scale_w):
    k, n = w_mat.shape
    m = x.shape[0]
    m_per = m // N_DEV
    n_streams = 2 * NSUB
    ns = n // n_streams

    def body(x_ref, w_ref, sx_ref, sw_ref, out_ref, *scratch):
        send_bufs = scratch[0:n_streams]
        comm_bufs = scratch[n_streams:2 * n_streams]
        send_sems = scratch[2 * n_streams:3 * n_streams]
        recv_sems = scratch[3 * n_streams:4 * n_streams]

        my = lax.axis_index("i")
        left = lax.rem(my + N_DEV - 1, N_DEV)
        right = lax.rem(my + 1, N_DEV)

        def tgt(st):
            return right if st < NSUB else left

        def send_chunk(d, s):
            off = N_DEV - 1 - s if d == 0 else 1 + s
            return lax.rem(my + 2 * N_DEV + off, N_DEV)

        def recv_chunk(d, s):
            off = N_DEV - 2 - s if d == 0 else 2 + s
            return lax.rem(my + 2 * N_DEV + off, N_DEV)

        w_st = [w_ref[:, st * ns:(st + 1) * ns].astype(jnp.bfloat16)
                for st in range(n_streams)]

        def xblk(c):
            return x_ref[pl.ds(c * m_per, m_per), :].astype(jnp.bfloat16)

        def partials(s):
            xa = xblk(recv_chunk(0, s))
            xb = xblk(recv_chunk(1, s))
            return [jnp.dot(xa if st < NSUB else xb, w_st[st],
                            preferred_element_type=jnp.float32)
                    for st in range(n_streams)]

        xa0 = xblk(send_chunk(0, 0))
        xb0 = xblk(send_chunk(1, 0))
        for st in range(n_streams):
            send_bufs[st][0, :, :] = jnp.dot(
                xa0 if st < NSUB else xb0, w_st[st],
                preferred_element_type=jnp.float32).astype(jnp.bfloat16)

        barrier_sem = pltpu.get_barrier_semaphore()
        for nbr in (left, right):
            pl.semaphore_signal(barrier_sem, inc=1, device_id=(nbr,),
                                device_id_type=pl.DeviceIdType.MESH)
        pl.semaphore_wait(barrier_sem, 2)

        def make_rdma(st, s):
            return pltpu.make_async_remote_copy(
                src_ref=send_bufs[st].at[s % 2],
                dst_ref=comm_bufs[st].at[s],
                send_sem=send_sems[st].at[s],
                recv_sem=recv_sems[st].at[s],
                device_id=(tgt(st),),
                device_id_type=pl.DeviceIdType.MESH,
            )

        rdmas = [[None] * (N_HOPS + 1) for _ in range(n_streams)]
        for st in range(n_streams):
            rdmas[st][0] = make_rdma(st, 0)
            rdmas[st][0].start()

        p = partials(0)

        order = []
        for j in range(NSUB):
            order += [j, NSUB + j]

        scale = sx_ref[0] * sw_ref[0]
        for s in range(N_HOPS):
            for st in order:
                rdmas[st][s].wait_recv()
                acc = comm_bufs[st][s, :, :].astype(jnp.float32) + p[st]
                if s < N_HOPS - 1:
                    if s >= 1:
                        rdmas[st][s - 1].wait_send()
                    send_bufs[st][(s + 1) % 2, :, :] = acc.astype(jnp.bfloat16)
                    rdmas[st][s + 1] = make_rdma(st, s + 1)
                    rdmas[st][s + 1].start()
                else:
                    out_ref[:, st * ns:(st + 1) * ns] = jnp.maximum(
                        acc * scale, 0.0)
            if s < N_HOPS - 1:
                p = partials(s + 1)

        for st in range(n_streams):
            rdmas[st][N_HOPS - 2].wait_send()
            rdmas[st][N_HOPS - 1].wait_send()

    return pl.pallas_call(
        body,
        out_shape=jax.ShapeDtypeStruct((m_per, n), jnp.float32),
        in_specs=[
            pl.BlockSpec(memory_space=pltpu.VMEM),
            pl.BlockSpec(memory_space=pltpu.VMEM),
            pl.BlockSpec(memory_space=pltpu.SMEM),
            pl.BlockSpec(memory_space=pltpu.SMEM),
        ],
        out_specs=pl.BlockSpec(memory_space=pltpu.VMEM),
        scratch_shapes=(
            [pltpu.VMEM((2, m_per, ns), jnp.bfloat16)] * n_streams +
            [pltpu.VMEM((N_HOPS, m_per, ns), jnp.bfloat16)] * n_streams +
            [pltpu.SemaphoreType.DMA((N_HOPS,))] * n_streams +
            [pltpu.SemaphoreType.DMA((N_HOPS,))] * n_streams
        ),
        compiler_params=pltpu.CompilerParams(collective_id=0),
    )(x, w_mat, scale_x, scale_w)
